# Initial kernel scaffold; baseline (speedup 1.0000x reference)
import jax
import jax.numpy as jnp
from jax.experimental import pallas as pl


def kernel(x, edge_index, edge_attr, batch, lin0_w, lin0_b, nn1_w, nn1_b, nn2_w, nn2_b, root_w, conv_b, gru_w_ih, gru_w_hh, gru_b_ih, gru_b_hh, lstm_w_ih, lstm_w_hh, lstm_b_ih, lstm_b_hh, lin1_w, lin1_b, lin2_w, lin2_b):
    raise NotImplementedError("write your pallas kernel here")


# speedup vs baseline: 2.5851x; 2.5851x over previous
import functools

import numpy as np

import jax
import jax.numpy as jnp
from jax import lax
from jax.experimental import pallas as pl
from jax.experimental.pallas import tpu as pltpu
from jax.experimental.pallas import tpu_sc as plsc

_N = 10000
_E = 160000
_ND = 128
_D = 32
_OD = 8
_B = 256

_NC = 2
_NS = 16
_NW = _NC * _NS
_EPW = _E // _NW
_CH = 40
_KJ = _EPW // _CH
_BLK = 200
_SUB = _BLK // _CH
_NBLK = _EPW // _BLK
_NPS = 624
_NTAIL = _N - _NPS * _NS


def _matmul_relu_body(x_ref, w_ref, b_ref, o_ref):
    o_ref[...] = jnp.maximum(
        jnp.dot(x_ref[...], w_ref[...], preferred_element_type=jnp.float32)
        + b_ref[...], 0.0)


def _matmul_relu(x, w, b, tile):
    m, k = x.shape
    n = w.shape[1]
    return pl.pallas_call(
        _matmul_relu_body,
        grid=(m // tile,),
        in_specs=[
            pl.BlockSpec((tile, k), lambda i: (i, 0)),
            pl.BlockSpec((k, n), lambda i: (0, 0)),
            pl.BlockSpec((1, n), lambda i: (0, 0)),
        ],
        out_specs=pl.BlockSpec((tile, n), lambda i: (i, 0)),
        out_shape=jax.ShapeDtypeStruct((m, n), jnp.float32),
    )(x, w, b)


def _lin0_body(x_ref, w_ref, b_ref, p_ref, o_ref, op_ref):
    o = jnp.maximum(
        jnp.dot(x_ref[...], w_ref[...], preferred_element_type=jnp.float32)
        + b_ref[...], 0.0)
    o_ref[...] = o
    op_ref[...] = jnp.dot(o, p_ref[...], preferred_element_type=jnp.float32)


def _lin0(x, w, b, pmat, tile=1000):
    return pl.pallas_call(
        _lin0_body,
        grid=(_N // tile,),
        in_specs=[
            pl.BlockSpec((tile, _ND), lambda i: (i, 0)),
            pl.BlockSpec((_ND, _D), lambda i: (0, 0)),
            pl.BlockSpec((1, _D), lambda i: (0, 0)),
            pl.BlockSpec((_D, _ND), lambda i: (0, 0)),
        ],
        out_specs=[pl.BlockSpec((tile, _D), lambda i: (i, 0)),
                   pl.BlockSpec((tile, _ND), lambda i: (i, 0))],
        out_shape=[jax.ShapeDtypeStruct((_N, _D), jnp.float32),
                   jax.ShapeDtypeStruct((_N, _ND), jnp.float32)],
    )(x, w, b, pmat)


def _msg_body(hid_ref, xg_ref, pt_ref, w2p_ref, r_ref, s_ref, b2_ref, o_ref):
    hid = hid_ref[...]
    xg = jnp.dot(xg_ref[...], pt_ref[...], preferred_element_type=jnp.float32)
    wg = jnp.dot(hid, w2p_ref[...], preferred_element_type=jnp.float32)
    xrep = jnp.dot(xg, r_ref[...], preferred_element_type=jnp.float32)
    acc = wg[:, 0:128] * xrep[:, 0:128]
    for g in range(1, 8):
        acc = acc + wg[:, g * 128:(g + 1) * 128] * xrep[:, g * 128:(g + 1) * 128]
    o_ref[...] = (
        jnp.dot(acc, s_ref[...], preferred_element_type=jnp.float32)
        + jnp.dot(xg, b2_ref[...], preferred_element_type=jnp.float32))


def _msg_compute(hid, xg128, pt, w2p, rmat, smat, b2, tile=640):
    return pl.pallas_call(
        _msg_body,
        grid=(_E // tile,),
        in_specs=[
            pl.BlockSpec((tile, _ND), lambda i: (i, 0)),
            pl.BlockSpec((tile, _ND), lambda i: (i, 0)),
            pl.BlockSpec((_ND, _D), lambda i: (0, 0)),
            pl.BlockSpec((_ND, 1024), lambda i: (0, 0)),
            pl.BlockSpec((_D, 1024), lambda i: (0, 0)),
            pl.BlockSpec((_ND, _ND), lambda i: (0, 0)),
            pl.BlockSpec((_D, _ND), lambda i: (0, 0)),
        ],
        out_specs=pl.BlockSpec((tile, _ND), lambda i: (i, 0)),
        out_shape=jax.ShapeDtypeStruct((_E, _ND), jnp.float32),
    )(hid, xg128, pt, w2p, rmat, smat, b2)


def _gru_body(a0_ref, a1_ref, out_ref, h_ref, rw_ref, cb_ref,
              wri_ref, wzi_ref, wni_ref, wrh_ref, wzh_ref, wnh_ref,
              bih_ref, bhh_ref, p_ref, pt_ref, o_ref, op_ref):
    out = out_ref[...]
    h = h_ref[...]
    agg = jnp.dot(a0_ref[...] + a1_ref[...], pt_ref[...],
                  preferred_element_type=jnp.float32)
    m = jnp.maximum(
        agg + jnp.dot(out, rw_ref[...], preferred_element_type=jnp.float32)
        + cb_ref[...], 0.0)
    gi_r = jnp.dot(m, wri_ref[...], preferred_element_type=jnp.float32) + bih_ref[0:1, :]
    gi_z = jnp.dot(m, wzi_ref[...], preferred_element_type=jnp.float32) + bih_ref[1:2, :]
    gi_n = jnp.dot(m, wni_ref[...], preferred_element_type=jnp.float32) + bih_ref[2:3, :]
    gh_r = jnp.dot(h, wrh_ref[...], preferred_element_type=jnp.float32) + bhh_ref[0:1, :]
    gh_z = jnp.dot(h, wzh_ref[...], preferred_element_type=jnp.float32) + bhh_ref[1:2, :]
    gh_n = jnp.dot(h, wnh_ref[...], preferred_element_type=jnp.float32) + bhh_ref[2:3, :]
    r = jax.nn.sigmoid(gi_r + gh_r)
    z = jax.nn.sigmoid(gi_z + gh_z)
    ncand = jnp.tanh(gi_n + r * gh_n)
    hn = (1.0 - z) * ncand + z * h
    o_ref[...] = hn
    op_ref[...] = jnp.dot(hn, p_ref[...], preferred_element_type=jnp.float32)


def _gru_step(a0, a1, out, h, rw, cb, wri, wzi, wni, wrh, wzh, wnh, bih, bhh,
              pmat, pt, tile=1000):
    node_spec = pl.BlockSpec((tile, _D), lambda i: (i, 0))
    wide_spec = pl.BlockSpec((tile, _ND), lambda i: (i, 0))
    w_spec = pl.BlockSpec((_D, _D), lambda i: (0, 0))
    return pl.pallas_call(
        _gru_body,
        grid=(_N // tile,),
        in_specs=[wide_spec, wide_spec, node_spec, node_spec,
                  w_spec, pl.BlockSpec((1, _D), lambda i: (0, 0)),
                  w_spec, w_spec, w_spec, w_spec, w_spec, w_spec,
                  pl.BlockSpec((3, _D), lambda i: (0, 0)),
                  pl.BlockSpec((3, _D), lambda i: (0, 0)),
                  pl.BlockSpec((_D, _ND), lambda i: (0, 0)),
                  pl.BlockSpec((_ND, _D), lambda i: (0, 0))],
        out_specs=[node_spec, pl.BlockSpec((tile, _ND), lambda i: (i, 0))],
        out_shape=[jax.ShapeDtypeStruct((_N, _D), jnp.float32),
                   jax.ShapeDtypeStruct((_N, _ND), jnp.float32)],
    )(a0, a1, out, h, rw, cb, wri, wzi, wni, wrh, wzh, wnh, bih, bhh, pmat, pt)


def _s2s_body(out_ref, batch_ref, wiq_ref, wir_ref, whh_ref, bih_ref, bhh_ref,
              l1q_ref, l1r_ref, l1b_ref, l2t_ref, l2b_ref, o_ref, e_scr):
    nt = 1000
    iota_b = lax.broadcasted_iota(jnp.int32, (1, _B), 1)
    qq = jnp.zeros((_B, _D), jnp.float32)
    qr = jnp.zeros((_B, _D), jnp.float32)
    hx = jnp.zeros((_B, _D), jnp.float32)
    cx = jnp.zeros((_B, _D), jnp.float32)
    ones_col = jnp.ones((nt, 1), jnp.float32)
    for _ in range(3):
        gates = []
        for g in range(4):
            pre = (
                jnp.dot(qq, wiq_ref[g * _D:(g + 1) * _D, :],
                        preferred_element_type=jnp.float32)
                + jnp.dot(qr, wir_ref[g * _D:(g + 1) * _D, :],
                          preferred_element_type=jnp.float32)
                + jnp.dot(hx, whh_ref[g * _D:(g + 1) * _D, :],
                          preferred_element_type=jnp.float32)
                + bih_ref[g:g + 1, :] + bhh_ref[g:g + 1, :])
            gates.append(pre)
        ig = jax.nn.sigmoid(gates[0])
        fg = jax.nn.sigmoid(gates[1])
        gg = jnp.tanh(gates[2])
        og = jax.nn.sigmoid(gates[3])
        cx = fg * cx + ig * gg
        hx = og * jnp.tanh(cx)
        q = hx
        e_max = jnp.full((1, _B), -1e30, jnp.float32)
        for t in range(_N // nt):
            out_t = out_ref[t * nt:(t + 1) * nt, :]
            et = lax.dot_general(out_t, q, (((1,), (1,)), ((), ())),
                                 preferred_element_type=jnp.float32)
            e_scr[t * nt:(t + 1) * nt, :] = et
            mask = batch_ref[t * nt:(t + 1) * nt, :] == iota_b
            e_max = jnp.maximum(
                e_max,
                jnp.max(jnp.where(mask, et, -1e30), axis=0, keepdims=True))
        den = jnp.zeros((_B, 1), jnp.float32)
        rn = jnp.zeros((_B, _D), jnp.float32)
        for t in range(_N // nt):
            out_t = out_ref[t * nt:(t + 1) * nt, :]
            et = e_scr[t * nt:(t + 1) * nt, :]
            mask = batch_ref[t * nt:(t + 1) * nt, :] == iota_b
            num = jnp.where(mask, jnp.exp(et - e_max), 0.0)
            den = den + lax.dot_general(num, ones_col, (((0,), (0,)), ((), ())),
                                        preferred_element_type=jnp.float32)
            rn = rn + lax.dot_general(num, out_t, (((0,), (0,)), ((), ())),
                                      preferred_element_type=jnp.float32)
        qr = rn / jnp.maximum(den, 1e-30)
        qq = q
    o1 = jnp.maximum(
        jnp.dot(qq, l1q_ref[...], preferred_element_type=jnp.float32)
        + jnp.dot(qr, l1r_ref[...], preferred_element_type=jnp.float32)
        + l1b_ref[...], 0.0)
    o_ref[...] = (jnp.dot(o1, l2t_ref[...], preferred_element_type=jnp.float32)
                  + l2b_ref[...])


def _set2set(out, batch2, wiq, wir, whh, bih, bhh, l1q, l1r, l1b, l2t, l2b):
    return pl.pallas_call(
        _s2s_body,
        out_shape=jax.ShapeDtypeStruct((_B, _OD), jnp.float32),
        scratch_shapes=[pltpu.VMEM((_N, _B), jnp.float32)],
    )(out, batch2, wiq, wir, whh, bih, bhh, l1q, l1r, l1b, l2t, l2b)


_MESH = plsc.VectorSubcoreMesh(core_axis_name="c", subcore_axis_name="s")


def _sc_gather(table128, idx3):
    @functools.partial(
        pl.kernel,
        out_type=jax.ShapeDtypeStruct((_E, _ND), jnp.float32),
        mesh=_MESH,
        scratch_types=[
            pltpu.VMEM((_KJ, _CH), jnp.int32),
            pltpu.VMEM((_BLK, _ND), jnp.float32),
            pltpu.SemaphoreType.DMA,
        ],
    )
    def k(table_hbm, idx_hbm, out_hbm, idx_v, rows_v, sem):
        c = lax.axis_index("c")
        s = lax.axis_index("s")
        w = s * _NC + c
        pltpu.sync_copy(idx_hbm.at[w], idx_v)
        base = w * _EPW

        def body(jj, carry):
            def sub(k, carry2):
                pltpu.async_copy(
                    table_hbm.at[idx_v.at[jj * _SUB + k]],
                    rows_v.at[pl.ds(k * _CH, _CH)], sem).wait()
                return carry2

            lax.fori_loop(0, _SUB, sub, 0)
            pltpu.sync_copy(rows_v, out_hbm.at[pl.ds(base + jj * _BLK, _BLK)])
            return carry

        lax.fori_loop(0, _NBLK, body, 0)

    return k(table128, idx3)


def _sc_scatter_add(msg, dst3, zeros_nd):
    @functools.partial(
        pl.kernel,
        out_type=jax.ShapeDtypeStruct((_NC, _N, _ND), jnp.float32),
        mesh=_MESH,
        scratch_types=[
            pltpu.VMEM_SHARED((_N, _ND), jnp.float32),
            pltpu.VMEM((_KJ, _CH), jnp.int32),
            pltpu.VMEM((_BLK, _ND), jnp.float32),
        ],
    )
    def k(msg_hbm, dst_hbm, zero_hbm, out_hbm, acc_sh, idx_v, msg_v):
        c = lax.axis_index("c")
        s = lax.axis_index("s")
        w = s * _NC + c
        pltpu.sync_copy(zero_hbm.at[pl.ds(s * _NPS, _NPS)],
                        acc_sh.at[pl.ds(s * _NPS, _NPS)])
        @pl.when(s == 0)
        def _():
            pltpu.sync_copy(zero_hbm.at[pl.ds(_NPS * _NS, _NTAIL)],
                            acc_sh.at[pl.ds(_NPS * _NS, _NTAIL)])
        plsc.subcore_barrier()
        pltpu.sync_copy(dst_hbm.at[w], idx_v)
        base = w * _EPW

        def body(jj, carry):
            pltpu.sync_copy(msg_hbm.at[pl.ds(base + jj * _BLK, _BLK)], msg_v)

            def sub(k, carry2):
                pltpu.sync_copy(msg_v.at[pl.ds(k * _CH, _CH)],
                                acc_sh.at[idx_v.at[jj * _SUB + k]], add=True)
                return carry2

            lax.fori_loop(0, _SUB, sub, 0)
            return carry

        lax.fori_loop(0, _NBLK, body, 0)
        plsc.subcore_barrier()
        pltpu.sync_copy(acc_sh.at[pl.ds(s * _NPS, _NPS)],
                        out_hbm.at[c].at[pl.ds(s * _NPS, _NPS)])
        @pl.when(s == 0)
        def _():
            pltpu.sync_copy(acc_sh.at[pl.ds(_NPS * _NS, _NTAIL)],
                            out_hbm.at[c].at[pl.ds(_NPS * _NS, _NTAIL)])

    return k(msg, dst3, zeros_nd)


def _build_msg_consts(nn2_w, nn2_b):
    l = np.arange(1024)
    g, r = l // 128, l % 128
    j, o = r // 32, r % 32
    perm = (4 * g + j) * 32 + o
    w2p = nn2_w.T[:, perm]
    rmat = np.zeros((_D, 1024), np.float32)
    rmat[4 * g + j, l] = 1.0
    smat = np.zeros((_ND, _ND), np.float32)
    smat[:, :_D] = np.tile(np.eye(_D, dtype=np.float32), (4, 1))
    b2 = jnp.concatenate(
        [nn2_b.reshape(_D, _D), jnp.zeros((_D, _ND - _D), jnp.float32)],
        axis=1)
    return w2p, jnp.asarray(rmat), jnp.asarray(smat), b2


def kernel(x, edge_index, edge_attr, batch, lin0_w, lin0_b, nn1_w, nn1_b,
           nn2_w, nn2_b, root_w, conv_b, gru_w_ih, gru_w_hh, gru_b_ih,
           gru_b_hh, lstm_w_ih, lstm_w_hh, lstm_b_ih, lstm_b_hh, lin1_w,
           lin1_b, lin2_w, lin2_b):
    src3 = edge_index[0].reshape(_NW, _KJ, _CH)
    dst3 = edge_index[1].reshape(_NW, _KJ, _CH)
    zeros_nd = jnp.zeros((_N, _ND), jnp.float32)
    pmat = jnp.concatenate(
        [jnp.eye(_D, dtype=jnp.float32),
         jnp.zeros((_D, _ND - _D), jnp.float32)], axis=1)

    out, out128 = _lin0(x, lin0_w.T, lin0_b.reshape(1, _D), pmat)
    hid = _matmul_relu(edge_attr, nn1_w.T, nn1_b.reshape(1, _ND), tile=5000)

    w2p, rmat, smat, b2 = _build_msg_consts(nn2_w, nn2_b)

    cb = conv_b.reshape(1, _D)
    wri = gru_w_ih[0:_D].T
    wzi = gru_w_ih[_D:2 * _D].T
    wni = gru_w_ih[2 * _D:3 * _D].T
    wrh = gru_w_hh[0:_D].T
    wzh = gru_w_hh[_D:2 * _D].T
    wnh = gru_w_hh[2 * _D:3 * _D].T
    bih = gru_b_ih.reshape(3, _D)
    bhh = gru_b_hh.reshape(3, _D)

    h = out
    for _ in range(3):
        xg128 = _sc_gather(out128, src3)
        msg = _msg_compute(hid, xg128, pmat.T, w2p, rmat, smat, b2)
        partials = _sc_scatter_add(msg, dst3, zeros_nd)
        h, out128 = _gru_step(partials[0], partials[1], out, h, root_w, cb,
                              wri, wzi, wni, wrh, wzh, wnh, bih, bhh,
                              pmat, pmat.T)
        out = h

    wiq = jnp.concatenate(
        [lstm_w_ih[g * _D:(g + 1) * _D, 0:_D].T for g in range(4)], axis=0)
    wir = jnp.concatenate(
        [lstm_w_ih[g * _D:(g + 1) * _D, _D:2 * _D].T for g in range(4)], axis=0)
    whh = jnp.concatenate(
        [lstm_w_hh[g * _D:(g + 1) * _D, :].T for g in range(4)], axis=0)
    bih4 = lstm_b_ih.reshape(4, _D)
    bhh4 = lstm_b_hh.reshape(4, _D)
    l1q = lin1_w[:, 0:_D].T
    l1r = lin1_w[:, _D:2 * _D].T
    l1b = lin1_b.reshape(1, _D)
    l2t = lin2_w.T
    l2b = lin2_b.reshape(1, _OD)
    batch2 = batch.reshape(_N, 1)

    return _set2set(out, batch2, wiq, wir, whh, bih4, bhh4,
                    l1q, l1r, l1b, l2t, l2b)


# speedup vs baseline: 2.7647x; 1.0695x over previous
import functools

import numpy as np

import jax
import jax.numpy as jnp
from jax import lax
from jax.experimental import pallas as pl
from jax.experimental.pallas import tpu as pltpu
from jax.experimental.pallas import tpu_sc as plsc

_N = 10000
_E = 160000
_ND = 128
_D = 32
_OD = 8
_B = 256

_NC = 2
_NS = 16
_NW = _NC * _NS
_EPW = _E // _NW
_CH = 40
_KJ = _EPW // _CH
_BLK = 200
_SUB = _BLK // _CH
_NBLK = _EPW // _BLK
_NPS = 624
_NTAIL = _N - _NPS * _NS


def _matmul_relu_body(x_ref, w_ref, b_ref, o_ref):
    o_ref[...] = jnp.maximum(
        jnp.dot(x_ref[...], w_ref[...], preferred_element_type=jnp.float32)
        + b_ref[...], 0.0)


def _matmul_relu(x, w, b, tile):
    m, k = x.shape
    n = w.shape[1]
    return pl.pallas_call(
        _matmul_relu_body,
        grid=(m // tile,),
        in_specs=[
            pl.BlockSpec((tile, k), lambda i: (i, 0)),
            pl.BlockSpec((k, n), lambda i: (0, 0)),
            pl.BlockSpec((1, n), lambda i: (0, 0)),
        ],
        out_specs=pl.BlockSpec((tile, n), lambda i: (i, 0)),
        out_shape=jax.ShapeDtypeStruct((m, n), jnp.float32),
    )(x, w, b)


def _lin0_body(x_ref, w_ref, b_ref, p_ref, o_ref, op_ref):
    o = jnp.maximum(
        jnp.dot(x_ref[...], w_ref[...], preferred_element_type=jnp.float32)
        + b_ref[...], 0.0)
    o_ref[...] = o
    op_ref[...] = jnp.dot(o, p_ref[...], preferred_element_type=jnp.float32)


def _lin0(x, w, b, pmat, tile=1000):
    return pl.pallas_call(
        _lin0_body,
        grid=(_N // tile,),
        in_specs=[
            pl.BlockSpec((tile, _ND), lambda i: (i, 0)),
            pl.BlockSpec((_ND, _D), lambda i: (0, 0)),
            pl.BlockSpec((1, _D), lambda i: (0, 0)),
            pl.BlockSpec((_D, _ND), lambda i: (0, 0)),
        ],
        out_specs=[pl.BlockSpec((tile, _D), lambda i: (i, 0)),
                   pl.BlockSpec((tile, _ND), lambda i: (i, 0))],
        out_shape=[jax.ShapeDtypeStruct((_N, _D), jnp.float32),
                   jax.ShapeDtypeStruct((_N, _ND), jnp.float32)],
    )(x, w, b, pmat)


def _msg_body(hid_ref, xg_ref, pt_ref, w2p_ref, r_ref, s_ref, b2_ref, o_ref):
    hid = hid_ref[...]
    xg = jnp.dot(xg_ref[...], pt_ref[...], preferred_element_type=jnp.float32)
    wg = jnp.dot(hid, w2p_ref[...], preferred_element_type=jnp.float32)
    xrep = jnp.dot(xg, r_ref[...], preferred_element_type=jnp.float32)
    acc = wg[:, 0:128] * xrep[:, 0:128]
    for g in range(1, 8):
        acc = acc + wg[:, g * 128:(g + 1) * 128] * xrep[:, g * 128:(g + 1) * 128]
    o_ref[...] = (
        jnp.dot(acc, s_ref[...], preferred_element_type=jnp.float32)
        + jnp.dot(xg, b2_ref[...], preferred_element_type=jnp.float32))


def _msg_compute(hid, xg128, pt, w2p, rmat, smat, b2, tile=640):
    return pl.pallas_call(
        _msg_body,
        grid=(_E // tile,),
        in_specs=[
            pl.BlockSpec((tile, _ND), lambda i: (i, 0)),
            pl.BlockSpec((tile, _ND), lambda i: (i, 0)),
            pl.BlockSpec((_ND, _D), lambda i: (0, 0)),
            pl.BlockSpec((_ND, 1024), lambda i: (0, 0)),
            pl.BlockSpec((_D, 1024), lambda i: (0, 0)),
            pl.BlockSpec((_ND, _ND), lambda i: (0, 0)),
            pl.BlockSpec((_D, _ND), lambda i: (0, 0)),
        ],
        out_specs=pl.BlockSpec((tile, _ND), lambda i: (i, 0)),
        out_shape=jax.ShapeDtypeStruct((_E, _ND), jnp.float32),
    )(hid, xg128, pt, w2p, rmat, smat, b2)


def _gru_body(a0_ref, a1_ref, out_ref, h_ref, rw_ref, cb_ref,
              wri_ref, wzi_ref, wni_ref, wrh_ref, wzh_ref, wnh_ref,
              bih_ref, bhh_ref, p_ref, pt_ref, o_ref, op_ref):
    out = out_ref[...]
    h = h_ref[...]
    agg = jnp.dot(a0_ref[...] + a1_ref[...], pt_ref[...],
                  preferred_element_type=jnp.float32)
    m = jnp.maximum(
        agg + jnp.dot(out, rw_ref[...], preferred_element_type=jnp.float32)
        + cb_ref[...], 0.0)
    gi_r = jnp.dot(m, wri_ref[...], preferred_element_type=jnp.float32) + bih_ref[0:1, :]
    gi_z = jnp.dot(m, wzi_ref[...], preferred_element_type=jnp.float32) + bih_ref[1:2, :]
    gi_n = jnp.dot(m, wni_ref[...], preferred_element_type=jnp.float32) + bih_ref[2:3, :]
    gh_r = jnp.dot(h, wrh_ref[...], preferred_element_type=jnp.float32) + bhh_ref[0:1, :]
    gh_z = jnp.dot(h, wzh_ref[...], preferred_element_type=jnp.float32) + bhh_ref[1:2, :]
    gh_n = jnp.dot(h, wnh_ref[...], preferred_element_type=jnp.float32) + bhh_ref[2:3, :]
    r = jax.nn.sigmoid(gi_r + gh_r)
    z = jax.nn.sigmoid(gi_z + gh_z)
    ncand = jnp.tanh(gi_n + r * gh_n)
    hn = (1.0 - z) * ncand + z * h
    o_ref[...] = hn
    op_ref[...] = jnp.dot(hn, p_ref[...], preferred_element_type=jnp.float32)


def _gru_step(a0, a1, out, h, rw, cb, wri, wzi, wni, wrh, wzh, wnh, bih, bhh,
              pmat, pt, tile=1000):
    node_spec = pl.BlockSpec((tile, _D), lambda i: (i, 0))
    wide_spec = pl.BlockSpec((tile, _ND), lambda i: (i, 0))
    w_spec = pl.BlockSpec((_D, _D), lambda i: (0, 0))
    return pl.pallas_call(
        _gru_body,
        grid=(_N // tile,),
        in_specs=[wide_spec, wide_spec, node_spec, node_spec,
                  w_spec, pl.BlockSpec((1, _D), lambda i: (0, 0)),
                  w_spec, w_spec, w_spec, w_spec, w_spec, w_spec,
                  pl.BlockSpec((3, _D), lambda i: (0, 0)),
                  pl.BlockSpec((3, _D), lambda i: (0, 0)),
                  pl.BlockSpec((_D, _ND), lambda i: (0, 0)),
                  pl.BlockSpec((_ND, _D), lambda i: (0, 0))],
        out_specs=[node_spec, pl.BlockSpec((tile, _ND), lambda i: (i, 0))],
        out_shape=[jax.ShapeDtypeStruct((_N, _D), jnp.float32),
                   jax.ShapeDtypeStruct((_N, _ND), jnp.float32)],
    )(a0, a1, out, h, rw, cb, wri, wzi, wni, wrh, wzh, wnh, bih, bhh, pmat, pt)


def _s2s_body(out_ref, batch_ref, wiq_ref, wir_ref, whh_ref, bih_ref, bhh_ref,
              l1q_ref, l1r_ref, l1b_ref, l2t_ref, l2b_ref, o_ref, e_scr):
    nt = 1000
    iota_b = lax.broadcasted_iota(jnp.int32, (1, _B), 1)
    qq = jnp.zeros((_B, _D), jnp.float32)
    qr = jnp.zeros((_B, _D), jnp.float32)
    hx = jnp.zeros((_B, _D), jnp.float32)
    cx = jnp.zeros((_B, _D), jnp.float32)
    ones_col = jnp.ones((nt, 1), jnp.float32)
    for _ in range(3):
        gates = []
        for g in range(4):
            pre = (
                jnp.dot(qq, wiq_ref[g * _D:(g + 1) * _D, :],
                        preferred_element_type=jnp.float32)
                + jnp.dot(qr, wir_ref[g * _D:(g + 1) * _D, :],
                          preferred_element_type=jnp.float32)
                + jnp.dot(hx, whh_ref[g * _D:(g + 1) * _D, :],
                          preferred_element_type=jnp.float32)
                + bih_ref[g:g + 1, :] + bhh_ref[g:g + 1, :])
            gates.append(pre)
        ig = jax.nn.sigmoid(gates[0])
        fg = jax.nn.sigmoid(gates[1])
        gg = jnp.tanh(gates[2])
        og = jax.nn.sigmoid(gates[3])
        cx = fg * cx + ig * gg
        hx = og * jnp.tanh(cx)
        q = hx
        e_max = jnp.full((1, _B), -1e30, jnp.float32)
        for t in range(_N // nt):
            out_t = out_ref[t * nt:(t + 1) * nt, :]
            et = lax.dot_general(out_t, q, (((1,), (1,)), ((), ())),
                                 preferred_element_type=jnp.float32)
            e_scr[t * nt:(t + 1) * nt, :] = et
            mask = batch_ref[t * nt:(t + 1) * nt, :] == iota_b
            e_max = jnp.maximum(
                e_max,
                jnp.max(jnp.where(mask, et, -1e30), axis=0, keepdims=True))
        den = jnp.zeros((_B, 1), jnp.float32)
        rn = jnp.zeros((_B, _D), jnp.float32)
        for t in range(_N // nt):
            out_t = out_ref[t * nt:(t + 1) * nt, :]
            et = e_scr[t * nt:(t + 1) * nt, :]
            mask = batch_ref[t * nt:(t + 1) * nt, :] == iota_b
            num = jnp.where(mask, jnp.exp(et - e_max), 0.0)
            den = den + lax.dot_general(num, ones_col, (((0,), (0,)), ((), ())),
                                        preferred_element_type=jnp.float32)
            rn = rn + lax.dot_general(num, out_t, (((0,), (0,)), ((), ())),
                                      preferred_element_type=jnp.float32)
        qr = rn / jnp.maximum(den, 1e-30)
        qq = q
    o1 = jnp.maximum(
        jnp.dot(qq, l1q_ref[...], preferred_element_type=jnp.float32)
        + jnp.dot(qr, l1r_ref[...], preferred_element_type=jnp.float32)
        + l1b_ref[...], 0.0)
    o_ref[...] = (jnp.dot(o1, l2t_ref[...], preferred_element_type=jnp.float32)
                  + l2b_ref[...])


def _set2set(out, batch2, wiq, wir, whh, bih, bhh, l1q, l1r, l1b, l2t, l2b):
    return pl.pallas_call(
        _s2s_body,
        out_shape=jax.ShapeDtypeStruct((_B, _OD), jnp.float32),
        scratch_shapes=[pltpu.VMEM((_N, _B), jnp.float32)],
    )(out, batch2, wiq, wir, whh, bih, bhh, l1q, l1r, l1b, l2t, l2b)


_MESH = plsc.VectorSubcoreMesh(core_axis_name="c", subcore_axis_name="s")


def _sc_gather(table128, idx3):
    @functools.partial(
        pl.kernel,
        out_type=jax.ShapeDtypeStruct((_E, _ND), jnp.float32),
        mesh=_MESH,
        scratch_types=[
            pltpu.VMEM((_KJ, _CH), jnp.int32),
            pltpu.VMEM((_BLK, _ND), jnp.float32),
            pltpu.VMEM((_BLK, _ND), jnp.float32),
            pltpu.SemaphoreType.DMA,
            pltpu.SemaphoreType.DMA,
        ],
    )
    def k(table_hbm, idx_hbm, out_hbm, idx_v, rows_a, rows_b, sem_a, sem_b):
        c = lax.axis_index("c")
        s = lax.axis_index("s")
        w = s * _NC + c
        pltpu.sync_copy(idx_hbm.at[w], idx_v)
        base = w * _EPW

        def fire(jj, buf, sem):
            for k in range(_SUB):
                pltpu.async_copy(table_hbm.at[idx_v.at[jj * _SUB + k]],
                                 buf.at[pl.ds(k * _CH, _CH)], sem)

        def drain(jj, buf, sem):
            for k in range(_SUB):
                pltpu.make_async_copy(table_hbm.at[idx_v.at[jj * _SUB + k]],
                                      buf.at[pl.ds(k * _CH, _CH)], sem).wait()

        def wb(jj, buf):
            pltpu.sync_copy(buf, out_hbm.at[pl.ds(base + jj * _BLK, _BLK)])

        fire(0, rows_a, sem_a)

        def body(t, carry):
            j0 = 2 * t
            fire(j0 + 1, rows_b, sem_b)
            drain(j0, rows_a, sem_a)
            wb(j0, rows_a)
            fire(j0 + 2, rows_a, sem_a)
            drain(j0 + 1, rows_b, sem_b)
            wb(j0 + 1, rows_b)
            return carry

        lax.fori_loop(0, (_NBLK - 1) // 2, body, 0)
        drain(_NBLK - 1, rows_a, sem_a)
        wb(_NBLK - 1, rows_a)

    return k(table128, idx3)


def _sc_scatter_add(msg, dst3, zeros_nd):
    @functools.partial(
        pl.kernel,
        out_type=jax.ShapeDtypeStruct((_NC, _N, _ND), jnp.float32),
        mesh=_MESH,
        scratch_types=[
            pltpu.VMEM_SHARED((_N, _ND), jnp.float32),
            pltpu.VMEM((_KJ, _CH), jnp.int32),
            pltpu.VMEM((_CH, _ND), jnp.float32),
            pltpu.VMEM((_CH, _ND), jnp.float32),
            pltpu.SemaphoreType.DMA,
            pltpu.SemaphoreType.DMA,
        ],
    )
    def k(msg_hbm, dst_hbm, zero_hbm, out_hbm, acc_sh, idx_v,
          msg_a, msg_b, sem_a, sem_b):
        c = lax.axis_index("c")
        s = lax.axis_index("s")
        w = s * _NC + c
        pltpu.sync_copy(zero_hbm.at[pl.ds(s * _NPS, _NPS)],
                        acc_sh.at[pl.ds(s * _NPS, _NPS)])
        @pl.when(s == 0)
        def _():
            pltpu.sync_copy(zero_hbm.at[pl.ds(_NPS * _NS, _NTAIL)],
                            acc_sh.at[pl.ds(_NPS * _NS, _NTAIL)])
        plsc.subcore_barrier()
        pltpu.sync_copy(dst_hbm.at[w], idx_v)
        base = w * _EPW

        def load(jj, buf):
            pltpu.sync_copy(msg_hbm.at[pl.ds(base + jj * _CH, _CH)], buf)

        def fire(jj, buf, sem):
            pltpu.async_copy(buf, acc_sh.at[idx_v.at[jj]], sem, add=True)

        def drain(jj, buf, sem):
            pltpu.make_async_copy(buf, acc_sh.at[idx_v.at[jj]], sem).wait()

        load(0, msg_a)

        def body(t, carry):
            j0 = 2 * t
            fire(j0, msg_a, sem_a)
            load(j0 + 1, msg_b)
            drain(j0, msg_a, sem_a)
            fire(j0 + 1, msg_b, sem_b)
            load(j0 + 2, msg_a)
            drain(j0 + 1, msg_b, sem_b)
            return carry

        lax.fori_loop(0, (_KJ - 1) // 2, body, 0)
        fire(_KJ - 1, msg_a, sem_a)
        drain(_KJ - 1, msg_a, sem_a)
        plsc.subcore_barrier()
        pltpu.sync_copy(acc_sh.at[pl.ds(s * _NPS, _NPS)],
                        out_hbm.at[c].at[pl.ds(s * _NPS, _NPS)])
        @pl.when(s == 0)
        def _():
            pltpu.sync_copy(acc_sh.at[pl.ds(_NPS * _NS, _NTAIL)],
                            out_hbm.at[c].at[pl.ds(_NPS * _NS, _NTAIL)])

    return k(msg, dst3, zeros_nd)


def _build_msg_consts(nn2_w, nn2_b):
    l = np.arange(1024)
    g, r = l // 128, l % 128
    j, o = r // 32, r % 32
    perm = (4 * g + j) * 32 + o
    w2p = nn2_w.T[:, perm]
    rmat = np.zeros((_D, 1024), np.float32)
    rmat[4 * g + j, l] = 1.0
    smat = np.zeros((_ND, _ND), np.float32)
    smat[:, :_D] = np.tile(np.eye(_D, dtype=np.float32), (4, 1))
    b2 = jnp.concatenate(
        [nn2_b.reshape(_D, _D), jnp.zeros((_D, _ND - _D), jnp.float32)],
        axis=1)
    return w2p, jnp.asarray(rmat), jnp.asarray(smat), b2


def kernel(x, edge_index, edge_attr, batch, lin0_w, lin0_b, nn1_w, nn1_b,
           nn2_w, nn2_b, root_w, conv_b, gru_w_ih, gru_w_hh, gru_b_ih,
           gru_b_hh, lstm_w_ih, lstm_w_hh, lstm_b_ih, lstm_b_hh, lin1_w,
           lin1_b, lin2_w, lin2_b):
    src3 = edge_index[0].reshape(_NW, _KJ, _CH)
    dst3 = edge_index[1].reshape(_NW, _KJ, _CH)
    zeros_nd = jnp.zeros((_N, _ND), jnp.float32)
    pmat = jnp.concatenate(
        [jnp.eye(_D, dtype=jnp.float32),
         jnp.zeros((_D, _ND - _D), jnp.float32)], axis=1)

    out, out128 = _lin0(x, lin0_w.T, lin0_b.reshape(1, _D), pmat)
    hid = _matmul_relu(edge_attr, nn1_w.T, nn1_b.reshape(1, _ND), tile=5000)

    w2p, rmat, smat, b2 = _build_msg_consts(nn2_w, nn2_b)

    cb = conv_b.reshape(1, _D)
    wri = gru_w_ih[0:_D].T
    wzi = gru_w_ih[_D:2 * _D].T
    wni = gru_w_ih[2 * _D:3 * _D].T
    wrh = gru_w_hh[0:_D].T
    wzh = gru_w_hh[_D:2 * _D].T
    wnh = gru_w_hh[2 * _D:3 * _D].T
    bih = gru_b_ih.reshape(3, _D)
    bhh = gru_b_hh.reshape(3, _D)

    h = out
    for _ in range(3):
        xg128 = _sc_gather(out128, src3)
        msg = _msg_compute(hid, xg128, pmat.T, w2p, rmat, smat, b2)
        partials = _sc_scatter_add(msg, dst3, zeros_nd)
        h, out128 = _gru_step(partials[0], partials[1], out, h, root_w, cb,
                              wri, wzi, wni, wrh, wzh, wnh, bih, bhh,
                              pmat, pmat.T)
        out = h

    wiq = jnp.concatenate(
        [lstm_w_ih[g * _D:(g + 1) * _D, 0:_D].T for g in range(4)], axis=0)
    wir = jnp.concatenate(
        [lstm_w_ih[g * _D:(g + 1) * _D, _D:2 * _D].T for g in range(4)], axis=0)
    whh = jnp.concatenate(
        [lstm_w_hh[g * _D:(g + 1) * _D, :].T for g in range(4)], axis=0)
    bih4 = lstm_b_ih.reshape(4, _D)
    bhh4 = lstm_b_hh.reshape(4, _D)
    l1q = lin1_w[:, 0:_D].T
    l1r = lin1_w[:, _D:2 * _D].T
    l1b = lin1_b.reshape(1, _D)
    l2t = lin2_w.T
    l2b = lin2_b.reshape(1, _OD)
    batch2 = batch.reshape(_N, 1)

    return _set2set(out, batch2, wiq, wir, whh, bih4, bhh4,
                    l1q, l1r, l1b, l2t, l2b)
